# baseline (speedup 1.0000x reference)
import jax
import jax.numpy as jnp
from jax.experimental import pallas as pl


def _moe_body(x_ref, rw_ref, wi_ref, wo_ref, out_ref):
    e = pl.program_id(1)
    x = x_ref[...]
    logits = jnp.dot(x, rw_ref[...], preferred_element_type=jnp.float32)
    mx = jnp.max(logits, axis=1, keepdims=True)
    p = 1.0 / jnp.sum(jnp.exp(logits - mx), axis=1)
    ncol = logits.shape[1]
    iota = jax.lax.broadcasted_iota(jnp.int32, logits.shape, 1)
    amax = jnp.min(jnp.where(logits == mx, iota, ncol), axis=1)
    w = jnp.where(amax == e, p, 0.0)
    h = jnp.maximum(jnp.dot(x, wi_ref[0], preferred_element_type=jnp.float32), 0.0)
    y = jnp.dot(h, wo_ref[0], preferred_element_type=jnp.float32)
    contrib = y * w[:, None]

    @pl.when(e == 0)
    def _():
        out_ref[...] = contrib

    @pl.when(e != 0)
    def _():
        out_ref[...] += contrib


def kernel(hidden_states, router_w, wi, wo):
    B, S, D = hidden_states.shape
    E, _, F = wi.shape
    T = B * S
    BLK = 512
    M = T // BLK
    x = hidden_states.reshape(T, D)

    out = pl.pallas_call(
        _moe_body,
        grid=(M, E),
        in_specs=[
            pl.BlockSpec((BLK, D), lambda m, e: (m, 0)),
            pl.BlockSpec((D, E), lambda m, e: (0, 0)),
            pl.BlockSpec((1, D, F), lambda m, e: (e, 0, 0)),
            pl.BlockSpec((1, F, D), lambda m, e: (e, 0, 0)),
        ],
        out_specs=pl.BlockSpec((BLK, D), lambda m, e: (m, 0)),
        out_shape=jax.ShapeDtypeStruct((T, D), jnp.float32),
    )(x, router_w, wi, wo)
    return out.reshape(B, S, D)


# speedup vs baseline: 2.2890x; 2.2890x over previous
import functools

import jax
import jax.numpy as jnp
from jax import lax
from jax.experimental import pallas as pl
from jax.experimental.pallas import tpu as pltpu
from jax.experimental.pallas import tpu_sc as plsc

BLK = 512
CH = 128
TPW = 256


def _router_body(x_ref, rw_ref, xs_ref, ei_ref, lp_ref, hist_ref):
    x = x_ref[...]
    logits = jnp.dot(x, rw_ref[...], preferred_element_type=jnp.float32)
    ncol = logits.shape[1]
    mx = jnp.max(logits, axis=1, keepdims=True)
    p = 1.0 / jnp.sum(jnp.exp(logits - mx), axis=1)
    iota_e = lax.broadcasted_iota(jnp.int32, logits.shape, 1)
    amax = jnp.min(jnp.where(logits == mx, iota_e, ncol), axis=1)
    onehot = (amax[:, None] == iota_e).astype(jnp.float32)
    ri = lax.broadcasted_iota(jnp.int32, (BLK, BLK), 0)
    ci = lax.broadcasted_iota(jnp.int32, (BLK, BLK), 1)
    tri = (ri > ci).astype(jnp.float32)
    ranks = jnp.dot(tri, onehot, preferred_element_type=jnp.float32)
    local_pos = jnp.sum(ranks * onehot, axis=1)
    xs_ref[...] = x * p[:, None]
    ei_ref[0, 0, :] = amax
    lp_ref[0, 0, :] = local_pos.astype(jnp.int32)
    hist_ref[0, 0, :] = jnp.sum(onehot, axis=0)


def _meta_body(nt_pad, hist_ref, ei_ref, lp_ref, pos_ref, tm_ref, te_ref,
               tf_ref, rl_ref, rh_ref):
    M, _, E = hist_ref.shape
    hist = hist_ref[...].reshape(M, E)

    def _lower_incl(n):
        ri = lax.broadcasted_iota(jnp.int32, (n, n), 0)
        ci = lax.broadcasted_iota(jnp.int32, (n, n), 1)
        return (ci <= ri).astype(jnp.float32)

    def _upper_incl(n):
        ri = lax.broadcasted_iota(jnp.int32, (n, n), 0)
        ci = lax.broadcasted_iota(jnp.int32, (n, n), 1)
        return (ri <= ci).astype(jnp.float32)

    col_cum = jnp.dot(_lower_incl(M), hist,
                      preferred_element_type=jnp.float32,
                      precision=lax.Precision.HIGHEST)
    col_prefix = col_cum - hist
    counts = jnp.sum(hist, axis=0, keepdims=True)
    c_end = jnp.dot(counts, _upper_incl(E),
                    preferred_element_type=jnp.float32,
                    precision=lax.Precision.HIGHEST)
    c_excl = c_end - counts
    base = c_excl + col_prefix

    nblk = ei_ref.shape[2]
    ei = ei_ref[...].reshape(M, nblk)
    lp = lp_ref[...].reshape(M, nblk)
    acc = jnp.zeros((M, nblk), jnp.float32)
    for e in range(E):
        acc = acc + jnp.where(ei == e, base[:, e:e + 1], 0.0)
    pos_ref[...] = (acc.astype(jnp.int32) + lp).reshape(M, 1, nblk)

    e_ge1 = lax.broadcasted_iota(jnp.int32, (M, E), 1) >= 1
    m_start = (lax.broadcasted_iota(jnp.int32, (M, E), 0) * BLK).astype(
        jnp.float32)
    ef = jnp.sum(((c_excl <= m_start) & e_ge1).astype(jnp.int32), axis=1)
    el = jnp.sum(((c_excl <= m_start + (BLK - 1)) & e_ge1).astype(jnp.int32),
                 axis=1)
    cnt = (el - ef + 1).reshape(1, M).astype(jnp.float32)
    st_incl = jnp.dot(cnt, _upper_incl(M),
                      preferred_element_type=jnp.float32,
                      precision=lax.Precision.HIGHEST)
    st = (st_incl - cnt).astype(jnp.int32)
    cnt = cnt.astype(jnp.int32)
    nt_act = jnp.sum(cnt)

    ti = lax.broadcasted_iota(jnp.int32, (nt_pad, M), 0)
    m_i = jnp.sum((st <= ti).astype(jnp.int32), axis=1) - 1
    onehot_m = (m_i[:, None] == lax.broadcasted_iota(
        jnp.int32, (nt_pad, M), 1)).astype(jnp.int32)
    ef_g = jnp.sum(onehot_m * ef[None, :], axis=1)
    st_g = jnp.sum(onehot_m * st, axis=1)
    i_vec = jnp.max(ti, axis=1)
    e_i = jnp.clip(ef_g + (i_vec - st_g), 0, E - 1)
    active = i_vec < nt_act
    first = ((i_vec == st_g) & active).astype(jnp.int32)
    onehot_e = (e_i[:, None] == lax.broadcasted_iota(
        jnp.int32, (nt_pad, E), 1)).astype(jnp.float32)
    ce_g = jnp.sum(onehot_e * c_excl, axis=1)
    cend_g = jnp.sum(onehot_e * c_end, axis=1)
    m_base = (m_i * BLK).astype(jnp.float32)
    lo = jnp.maximum(ce_g, m_base) - m_base
    hi = jnp.minimum(cend_g, m_base + BLK) - m_base
    lo = jnp.where(active, lo, 0.0).astype(jnp.int32)
    hi = jnp.where(active, hi, 0.0).astype(jnp.int32)
    tm_ref[...] = m_i.reshape(1, nt_pad)
    te_ref[...] = e_i.reshape(1, nt_pad)
    tf_ref[...] = first.reshape(1, nt_pad)
    rl_ref[...] = lo.reshape(1, nt_pad)
    rh_ref[...] = hi.reshape(1, nt_pad)


def _ffn_body(tm_ref, te_ref, tf_ref, rl_ref, rh_ref,
              x_ref, wi_ref, wo_ref, out_ref):
    i = pl.program_id(0)
    lo = rl_ref[0, i]
    hi = rh_ref[0, i]
    first = tf_ref[0, i]
    r = lax.broadcasted_iota(jnp.int32, (BLK, 1), 0)
    mask = (r >= lo) & (r < hi)
    x = x_ref[...]
    h = jnp.dot(x, wi_ref[0], preferred_element_type=jnp.float32)
    h = jnp.where(mask, jnp.maximum(h, 0.0), 0.0)
    y = jnp.dot(h, wo_ref[0], preferred_element_type=jnp.float32)

    @pl.when(first == 1)
    def _():
        out_ref[...] = y

    @pl.when(first == 0)
    def _():
        out_ref[...] += y


def _make_sc_dispatch(T, D):
    mesh = plsc.VectorSubcoreMesh(core_axis_name="c", subcore_axis_name="s")

    @functools.partial(
        pl.kernel,
        mesh=mesh,
        out_type=jax.ShapeDtypeStruct((T, D), jnp.float32),
        scratch_types=[
            pltpu.VMEM((CH,), jnp.int32),
            pltpu.VMEM((CH, D), jnp.float32),
            pltpu.SemaphoreType.DMA,
        ],
    )
    def dispatch(x_hbm, pos_hbm, xs_hbm, idx_v, rows_v, sem):
        wid = lax.axis_index("s") * 2 + lax.axis_index("c")
        for ch in range(TPW // CH):
            t0 = wid * TPW + ch * CH
            pltpu.sync_copy(pos_hbm.at[pl.ds(t0, CH)], idx_v)
            pltpu.sync_copy(x_hbm.at[pl.ds(t0, CH)], rows_v)
            pltpu.async_copy(rows_v, xs_hbm.at[idx_v], sem).wait()

    return dispatch


def _make_sc_combine(T, D):
    mesh = plsc.VectorSubcoreMesh(core_axis_name="c", subcore_axis_name="s")

    @functools.partial(
        pl.kernel,
        mesh=mesh,
        out_type=jax.ShapeDtypeStruct((T, D), jnp.float32),
        scratch_types=[
            pltpu.VMEM((CH,), jnp.int32),
            pltpu.VMEM((CH, D), jnp.float32),
            pltpu.SemaphoreType.DMA,
        ],
    )
    def combine(os_hbm, pos_hbm, out_hbm, idx_v, rows_v, sem):
        wid = lax.axis_index("s") * 2 + lax.axis_index("c")
        for ch in range(TPW // CH):
            t0 = wid * TPW + ch * CH
            pltpu.sync_copy(pos_hbm.at[pl.ds(t0, CH)], idx_v)
            pltpu.async_copy(os_hbm.at[idx_v], rows_v, sem).wait()
            pltpu.sync_copy(rows_v, out_hbm.at[pl.ds(t0, CH)])

    return combine


def kernel(hidden_states, router_w, wi, wo):
    B, S, D = hidden_states.shape
    E, _, F = wi.shape
    T = B * S
    M = T // BLK
    NT = M + E - 1
    NT_PAD = ((NT + 7) // 8) * 8
    x = hidden_states.reshape(T, D)

    x_scaled, ei3, lp3, hist3 = pl.pallas_call(
        _router_body,
        grid=(M,),
        in_specs=[
            pl.BlockSpec((BLK, D), lambda m: (m, 0)),
            pl.BlockSpec((D, E), lambda m: (0, 0)),
        ],
        out_specs=[
            pl.BlockSpec((BLK, D), lambda m: (m, 0)),
            pl.BlockSpec((1, 1, BLK), lambda m: (m, 0, 0)),
            pl.BlockSpec((1, 1, BLK), lambda m: (m, 0, 0)),
            pl.BlockSpec((1, 1, E), lambda m: (m, 0, 0)),
        ],
        out_shape=[
            jax.ShapeDtypeStruct((T, D), jnp.float32),
            jax.ShapeDtypeStruct((M, 1, BLK), jnp.int32),
            jax.ShapeDtypeStruct((M, 1, BLK), jnp.int32),
            jax.ShapeDtypeStruct((M, 1, E), jnp.float32),
        ],
    )(x, router_w)

    pos3, tm, te, tf, rl, rh = pl.pallas_call(
        functools.partial(_meta_body, NT_PAD),
        out_shape=[
            jax.ShapeDtypeStruct((M, 1, BLK), jnp.int32),
            jax.ShapeDtypeStruct((1, NT_PAD), jnp.int32),
            jax.ShapeDtypeStruct((1, NT_PAD), jnp.int32),
            jax.ShapeDtypeStruct((1, NT_PAD), jnp.int32),
            jax.ShapeDtypeStruct((1, NT_PAD), jnp.int32),
            jax.ShapeDtypeStruct((1, NT_PAD), jnp.int32),
        ],
    )(hist3, ei3, lp3)

    pos = pos3.reshape(T)

    x_sorted = _make_sc_dispatch(T, D)(x_scaled, pos)

    out_sorted = pl.pallas_call(
        _ffn_body,
        grid_spec=pltpu.PrefetchScalarGridSpec(
            num_scalar_prefetch=5,
            grid=(NT_PAD,),
            in_specs=[
                pl.BlockSpec((BLK, D),
                             lambda i, tm, te, tf, rl, rh: (tm[0, i], 0)),
                pl.BlockSpec((1, D, F),
                             lambda i, tm, te, tf, rl, rh: (te[0, i], 0, 0)),
                pl.BlockSpec((1, F, D),
                             lambda i, tm, te, tf, rl, rh: (te[0, i], 0, 0)),
            ],
            out_specs=pl.BlockSpec(
                (BLK, D), lambda i, tm, te, tf, rl, rh: (tm[0, i], 0)),
        ),
        out_shape=jax.ShapeDtypeStruct((T, D), jnp.float32),
    )(tm, te, tf, rl, rh, x_sorted, wi, wo)

    out = _make_sc_combine(T, D)(out_sorted, pos)
    return out.reshape(B, S, D)
